# baseline (speedup 1.0000x reference)
import functools
import jax
import jax.numpy as jnp
from jax import lax
from jax.experimental import pallas as pl
from jax.experimental.pallas import tpu as pltpu
from jax.experimental.pallas import tpu_sc as plsc

_N = 10000
_E = 320000
_D = 128
_NC = 2
_NS = 16
_CH = 80
_EPT = _E // _NS
_RPT = _N // _NS
_ACCW = 208
_VOX = 8
_K = _VOX * _VOX
_KK = _K * _K
_NEI = _KK - _K


def _iota16():
    return lax.iota(jnp.int32, 16)


_ZR = 624
_BCH = 50
_BE = _BCH * _CH
_NBLK = _EPT // _BE


def _edge_body(x_hbm, ei_hbm, px_hbm, py_hbm, zm_hbm, zs_hbm,
               outm_hbm, outs_hbm, out1_hbm,
               srcb, dstb, wv,
               dstl0, dstl1, pjx0, pjy0, pix0, piy0, pjx1, pjy1, pix1, piy1,
               rows0, rows1, outs0, outs1,
               acc_sh, accs_sh, sg0, sg1, ss0, ss1):
    c = lax.axis_index("c")
    s = lax.axis_index("s")
    a0 = (1 - c).astype(jnp.float32)
    a1 = c.astype(jnp.float32)
    iot = _iota16()
    m_is0 = iot == 0
    sv1 = jnp.where(iot == 1, 1.0, 0.0).astype(jnp.float32)

    slots = (
        dict(dstl=dstl0, pjx=pjx0, pjy=pjy0, pix=pix0, piy=piy0,
             rows=rows0, outs=outs0, sg=sg0, ss=ss0),
        dict(dstl=dstl1, pjx=pjx1, pjy=pjy1, pix=pix1, piy=piy1,
             rows=rows1, outs=outs1, sg=sg1, ss=ss1),
    )

    def g_descs(kb, sl, pass1):
        srcs = srcb.at[pl.ds(kb * _CH, _CH)]
        dsts = dstb.at[pl.ds(kb * _CH, _CH)]
        ds_ = [(x_hbm.at[srcs], sl["rows"])]
        if pass1:
            ds_ += [(px_hbm.at[srcs], sl["pjx"]),
                    (py_hbm.at[srcs], sl["pjy"]),
                    (px_hbm.at[dsts], sl["pix"]),
                    (py_hbm.at[dsts], sl["piy"])]
        return ds_

    def fire_gathers(kb, sl, pass1):
        for src, dst in g_descs(kb, sl, pass1):
            pltpu.async_copy(src, dst, sl["sg"])

    def wait_gathers(kb, sl, pass1):
        for src, dst in g_descs(kb, sl, pass1):
            pltpu.make_async_copy(src, dst, sl["sg"]).wait()

    def fire_scatter(sl, pass1):
        pltpu.async_copy(sl["rows"], acc_sh.at[sl["dstl"]], sl["ss"],
                         add=True)
        if pass1:
            pltpu.async_copy(sl["outs"], accs_sh.at[sl["dstl"]], sl["ss"],
                             add=True)

    def wait_scatter(sl, pass1):
        pltpu.make_async_copy(sl["rows"], acc_sh.at[sl["dstl"]],
                              sl["ss"]).wait()
        if pass1:
            pltpu.make_async_copy(sl["outs"], accs_sh.at[sl["dstl"]],
                                  sl["ss"]).wait()

    def compute_chunk(kb, sl):
        for g in range(_CH // 16):
            vsl = pl.ds(g * 16, 16)
            dx = sl["pix"][vsl] - sl["pjx"][vsl]
            dy = sl["piy"][vsl] - sl["pjy"][vsl]
            sc = 1.0 / (dx * dx + dy * dy + 0.01)
            wv[vsl] = (dx * a0 + dy * a1) * sc
            sl["dstl"][vsl] = dstb[pl.ds(kb * _CH + g * 16, 16)]

        rows, outs = sl["rows"], sl["outs"]

        def edge2(e2, _):
            for u in (0, 1):
                e = e2 * 2 + u
                wf = plsc.load_gather(wv, [jnp.broadcast_to(e, (16,))])
                for j in range(_D // 16):
                    rows[e, pl.ds(j * 16, 16)] = \
                        wf * rows[e, pl.ds(j * 16, 16)]
                outs[e, pl.ds(0, 16)] = jnp.where(m_is0, wf, sv1)
            return 0

        lax.fori_loop(0, _CH // 2, edge2, 0)

    def copy_dstl(kb, sl):
        for g in range(_CH // 16):
            vsl = pl.ds(g * 16, 16)
            sl["dstl"][vsl] = dstb[pl.ds(kb * _CH + g * 16, 16)]

    def run_pass(pass1):
        pltpu.sync_copy(zm_hbm, acc_sh.at[pl.ds(s * _ZR, _ZR)])
        if pass1:
            pltpu.sync_copy(zs_hbm, accs_sh.at[pl.ds(s * _ZR, _ZR)])

        @pl.when(s == _NS - 1)
        def ztail():
            pltpu.sync_copy(zm_hbm.at[pl.ds(0, 16)],
                            acc_sh.at[pl.ds(_NS * _ZR, 16)])
            if pass1:
                pltpu.sync_copy(zs_hbm.at[pl.ds(0, 16)],
                                accs_sh.at[pl.ds(_NS * _ZR, 16)])
        plsc.subcore_barrier()

        for b in range(_NBLK):
            base = s * _EPT + b * _BE
            pltpu.sync_copy(ei_hbm.at[0, pl.ds(base, _BE)], srcb)
            pltpu.sync_copy(ei_hbm.at[1, pl.ds(base, _BE)], dstb)
            fire_gathers(0, slots[0], pass1)

            def pair(i2, _):
                for par in (0, 1):
                    kb = i2 * 2 + par
                    sl = slots[par]
                    nxt = slots[1 - par]
                    if par == 0:
                        @pl.when(i2 > 0)
                        def wn():
                            wait_scatter(nxt, pass1)
                        fire_gathers(kb + 1, nxt, pass1)
                    else:
                        wait_scatter(nxt, pass1)

                        @pl.when(i2 < _BCH // 2 - 1)
                        def pf():
                            fire_gathers(kb + 1, nxt, pass1)
                    wait_gathers(kb, sl, pass1)
                    if pass1:
                        compute_chunk(kb, sl)
                    else:
                        copy_dstl(kb, sl)
                    fire_scatter(sl, pass1)
                return 0

            lax.fori_loop(0, _BCH // 2, pair, 0)
            wait_scatter(slots[1], pass1)
        plsc.subcore_barrier()

    run_pass(True)
    obase = c * _N
    pltpu.sync_copy(acc_sh.at[pl.ds(s * _ZR, _ZR)],
                    outm_hbm.at[pl.ds(obase + s * _ZR, _ZR)])
    pltpu.sync_copy(accs_sh.at[pl.ds(s * _ZR, _ZR)],
                    outs_hbm.at[pl.ds(obase + s * _ZR, _ZR)])

    @pl.when(s == _NS - 1)
    def otail1():
        pltpu.sync_copy(acc_sh.at[pl.ds(_NS * _ZR, 16)],
                        outm_hbm.at[pl.ds(obase + _NS * _ZR, 16)])
        pltpu.sync_copy(accs_sh.at[pl.ds(_NS * _ZR, 16)],
                        outs_hbm.at[pl.ds(obase + _NS * _ZR, 16)])
    plsc.subcore_barrier()

    run_pass(False)

    @pl.when(c == 0)
    def otail2():
        pltpu.sync_copy(acc_sh.at[pl.ds(s * _ZR, _ZR)],
                        out1_hbm.at[pl.ds(s * _ZR, _ZR)])

        @pl.when(s == _NS - 1)
        def otail2t():
            pltpu.sync_copy(acc_sh.at[pl.ds(_NS * _ZR, 16)],
                            out1_hbm.at[pl.ds(_NS * _ZR, 16)])


def _edge_stage(x, ei, px, py):
    mesh = plsc.VectorSubcoreMesh(core_axis_name="c", subcore_axis_name="s",
                                  num_cores=_NC, num_subcores=_NS)
    zm = jnp.zeros((_ZR, _D), jnp.float32)
    zs = jnp.zeros((_ZR, 16), jnp.float32)
    f = pl.kernel(
        _edge_body,
        out_type=(
            jax.ShapeDtypeStruct((_NC * _N, _D), jnp.float32),
            jax.ShapeDtypeStruct((_NC * _N, 16), jnp.float32),
            jax.ShapeDtypeStruct((_N, _D), jnp.float32),
        ),
        mesh=mesh,
        compiler_params=pltpu.CompilerParams(
            use_tc_tiling_on_sc=False, needs_layout_passes=False),
        scratch_types=[
            pltpu.VMEM((_BE,), jnp.int32),
            pltpu.VMEM((_BE,), jnp.int32),
            pltpu.VMEM((_CH,), jnp.float32),
            pltpu.VMEM((_CH,), jnp.int32),
            pltpu.VMEM((_CH,), jnp.int32),
            pltpu.VMEM((_CH,), jnp.float32),
            pltpu.VMEM((_CH,), jnp.float32),
            pltpu.VMEM((_CH,), jnp.float32),
            pltpu.VMEM((_CH,), jnp.float32),
            pltpu.VMEM((_CH,), jnp.float32),
            pltpu.VMEM((_CH,), jnp.float32),
            pltpu.VMEM((_CH,), jnp.float32),
            pltpu.VMEM((_CH,), jnp.float32),
            pltpu.VMEM((_CH, _D), jnp.float32),
            pltpu.VMEM((_CH, _D), jnp.float32),
            pltpu.VMEM((_CH, 16), jnp.float32),
            pltpu.VMEM((_CH, 16), jnp.float32),
            pltpu.VMEM_SHARED((_N, _D), jnp.float32),
            pltpu.VMEM_SHARED((_N, 16), jnp.float32),
            pltpu.SemaphoreType.DMA,
            pltpu.SemaphoreType.DMA,
            pltpu.SemaphoreType.DMA,
            pltpu.SemaphoreType.DMA,
        ],
    )
    return f(x, ei, px, py, zm, zs)


def _mlp_body(sa_ref, sb_ref, c0_ref, c1_ref, s1_ref, x_ref, wt_ref, b_ref,
              out_ref):
    sa = sa_ref[...]
    sb = sb_ref[...]
    s1c = s1_ref[...]
    xb = x_ref[...]
    suma = c0_ref[:, 0:1]
    sumb = c1_ref[:, 0:1]
    cnt = jnp.maximum(c0_ref[:, 1:2], 1.0)
    prop = jnp.concatenate(
        [(xb * suma - sa) / cnt, (xb * sumb - sb) / cnt, s1c / cnt, xb], axis=1)
    acc = jnp.dot(prop, wt_ref[...], preferred_element_type=jnp.float32)
    out_ref[...] = jnp.maximum(acc + b_ref[...], 0.0)


def _mlp_stage(soutm, souts, sout1, x, wt, b2):
    blk = 1000
    grid = _N // blk
    return pl.pallas_call(
        _mlp_body,
        grid=(grid,),
        in_specs=[
            pl.BlockSpec((blk, _D), lambda i: (i, 0)),
            pl.BlockSpec((blk, _D), lambda i: (grid + i, 0)),
            pl.BlockSpec((blk, 16), lambda i: (i, 0)),
            pl.BlockSpec((blk, 16), lambda i: (grid + i, 0)),
            pl.BlockSpec((blk, _D), lambda i: (i, 0)),
            pl.BlockSpec((blk, _D), lambda i: (i, 0)),
            pl.BlockSpec((4 * _D, _D), lambda i: (0, 0)),
            pl.BlockSpec((1, _D), lambda i: (0, 0)),
        ],
        out_specs=pl.BlockSpec((blk, _D), lambda i: (i, 0)),
        out_shape=jax.ShapeDtypeStruct((_N, _D), jnp.float32),
    )(soutm, soutm, souts, souts, sout1, x, wt, b2)


def _pool_body(xc_hbm, px_hbm, py_hbm, ei_hbm, outx_hbm, outpos_hbm,
               outei_hbm, posx, posy, xcb, posxc, posyc, srcv, dstv, occ,
               maxf, pkl, stripv, redv, stripx, redx, occv, codesb, eib,
               pkv, pposb, occ_sh, occr_sh, max_sh, pk_sh, sem):
    c = lax.axis_index("c")
    s = lax.axis_index("s")
    iot = _iota16()
    onesf = jnp.full((16,), 1.0, jnp.float32)
    m0 = iot == 0

    def vox(p):
        return jnp.clip((p * 8.0).astype(jnp.int32), 0, _VOX - 1)

    @pl.when(c == 0)
    def occupancy():
        pltpu.sync_copy(px_hbm, posx)
        pltpu.sync_copy(py_hbm, posy)

        def z(g, _):
            occ[pl.ds(g * 16, 16)] = jnp.zeros((16,), jnp.float32)
            return 0
        lax.fori_loop(0, _KK // 16, z, 0)

        for b in range(_EPT // _BE):
            base = s * _EPT + b * _BE
            pltpu.sync_copy(ei_hbm.at[0, pl.ds(base, _BE)], srcv)
            pltpu.sync_copy(ei_hbm.at[1, pl.ds(base, _BE)], dstv)

            def grp(g, _):
                vsl = pl.ds(g * 16, 16)
                s16 = srcv[vsl]
                d16 = dstv[vsl]
                cs = vox(plsc.load_gather(posx, [s16])) * _VOX + \
                    vox(plsc.load_gather(posy, [s16]))
                cd = vox(plsc.load_gather(posx, [d16])) * _VOX + \
                    vox(plsc.load_gather(posy, [d16]))
                code = cs * _K + cd
                plsc.store_scatter(occ, [code], onesf, mask=cs != cd)
                return 0
            lax.fori_loop(0, _BE // 16, grp, 0)

        pltpu.sync_copy(occ, occ_sh.at[s])
        plsc.subcore_barrier()
        strip = _KK // _NS
        pltpu.sync_copy(occ_sh.at[:, pl.ds(s * strip, strip)], stripv)
        for k in range(strip // 16):
            accv = stripv[0, pl.ds(k * 16, 16)]
            for r in range(1, _NS):
                accv = jnp.maximum(accv, stripv[r, pl.ds(k * 16, 16)])
            redv[pl.ds(k * 16, 16)] = accv
        pltpu.sync_copy(redv, occr_sh.at[pl.ds(s * strip, strip)])
        plsc.subcore_barrier()

        @pl.when(s == 0)
        def compact():
            pltpu.sync_copy(occr_sh, occv)

            def fill(g, _):
                codesb[pl.ds(g * 16, 16)] = jnp.full((16,), _KK, jnp.int32)
                return 0
            lax.fori_loop(0, (_NEI + 16) // 16, fill, 0)

            def step(g, off):
                ov = occv[pl.ds(g * 16, 16)]
                m = ov > 0.0
                codes = jnp.broadcast_to(g * 16, (16,)) + iot
                plsc.store_compressed(codesb.at[pl.ds(off, 16)], codes, mask=m)
                return off + jnp.max(plsc.all_reduce_population_count(m))
            lax.fori_loop(0, _KK // 16, step, 0)

            def emit(g, _):
                cv = codesb[pl.ds(g * 16, 16)]
                eib[0, pl.ds(g * 16, 16)] = lax.shift_right_arithmetic(cv, 6)
                eib[1, pl.ds(g * 16, 16)] = lax.bitwise_and(cv, 63)
                return 0
            lax.fori_loop(0, _NEI // 16, emit, 0)
            pltpu.sync_copy(eib, outei_hbm)

    @pl.when(c == 1)
    def pooling():
        def ini(g, _):
            maxf[pl.ds(g * 16, 16)] = jnp.full((16,), -jnp.inf, jnp.float32)
            return 0
        lax.fori_loop(0, (_K * _D) // 16, ini, 0)
        for k in range(192 // 16):
            pkl[pl.ds(k * 16, 16)] = jnp.zeros((16,), jnp.float32)

        def do_chunk(base, nrows):
            pltpu.sync_copy(xc_hbm.at[pl.ds(base, nrows)],
                            xcb.at[pl.ds(0, nrows)])
            pltpu.sync_copy(px_hbm.at[pl.ds(base, nrows)],
                            posxc.at[pl.ds(0, nrows)])
            pltpu.sync_copy(py_hbm.at[pl.ds(base, nrows)],
                            posyc.at[pl.ds(0, nrows)])

            def row(r, _):
                rsp = jnp.broadcast_to(r, (16,))
                px = plsc.load_gather(posxc, [rsp])
                py = plsc.load_gather(posyc, [rsp])
                code = vox(px) * _VOX + vox(py)
                for j in range(_D // 16):
                    idx = code * _D + j * 16 + iot
                    cur = plsc.load_gather(maxf, [idx])
                    xv = xcb[r, pl.ds(j * 16, 16)]
                    plsc.store_scatter(maxf, [idx], jnp.maximum(cur, xv))
                plsc.addupdate_scatter(pkl, [code], px, mask=m0)
                plsc.addupdate_scatter(pkl, [code + _K], py, mask=m0)
                plsc.addupdate_scatter(pkl, [code + 2 * _K], onesf, mask=m0)
                return 0
            lax.fori_loop(0, nrows, row, 0)

        for j in range(6):
            do_chunk(s * 624 + j * 104, 104)

        @pl.when(s == 0)
        def rtail():
            do_chunk(_NS * 624, 16)

        pltpu.sync_copy(maxf, max_sh.at[s])
        pltpu.sync_copy(pkl, pk_sh.at[s])
        plsc.subcore_barrier()
        strip = (_K * _D) // _NS
        pltpu.sync_copy(max_sh.at[:, pl.ds(s * strip, strip)], stripx)
        for k in range(strip // 16):
            accv = stripx[0, pl.ds(k * 16, 16)]
            for r in range(1, _NS):
                accv = jnp.maximum(accv, stripx[r, pl.ds(k * 16, 16)])
            redx[k // 8, pl.ds((k % 8) * 16, 16)] = accv
        pltpu.sync_copy(redx, outx_hbm.at[pl.ds(s * 4, 4)])

        @pl.when(s == 15)
        def posfin():
            pltpu.sync_copy(pk_sh, pkv)
            for k in range(4):
                pxv = pkv[0, pl.ds(k * 16, 16)]
                pyv = pkv[0, pl.ds(_K + k * 16, 16)]
                cnv = pkv[0, pl.ds(2 * _K + k * 16, 16)]
                for r in range(1, _NS):
                    pxv = pxv + pkv[r, pl.ds(k * 16, 16)]
                    pyv = pyv + pkv[r, pl.ds(_K + k * 16, 16)]
                    cnv = cnv + pkv[r, pl.ds(2 * _K + k * 16, 16)]
                cnv = jnp.maximum(cnv, 1.0)
                ridx = (iot + k * 16) * 2
                plsc.store_scatter(pposb, [ridx], pxv / cnv)
                plsc.store_scatter(pposb, [ridx + 1], pyv / cnv)
            pltpu.sync_copy(pposb, outpos_hbm)


def _pool_stage(xc, px, py, ei):
    mesh = plsc.VectorSubcoreMesh(core_axis_name="c", subcore_axis_name="s",
                                  num_cores=_NC, num_subcores=_NS)
    f = pl.kernel(
        _pool_body,
        compiler_params=pltpu.CompilerParams(
            use_tc_tiling_on_sc=False, needs_layout_passes=False),
        out_type=(
            jax.ShapeDtypeStruct((_K, _D), jnp.float32),
            jax.ShapeDtypeStruct((2 * _K,), jnp.float32),
            jax.ShapeDtypeStruct((2, _NEI), jnp.int32),
        ),
        mesh=mesh,
        scratch_types=[
            pltpu.VMEM((_N,), jnp.float32),
            pltpu.VMEM((_N,), jnp.float32),
            pltpu.VMEM((104, _D), jnp.float32),
            pltpu.VMEM((104,), jnp.float32),
            pltpu.VMEM((104,), jnp.float32),
            pltpu.VMEM((_BE,), jnp.int32),
            pltpu.VMEM((_BE,), jnp.int32),
            pltpu.VMEM((_KK,), jnp.float32),
            pltpu.VMEM((_K * _D,), jnp.float32),
            pltpu.VMEM((3 * _K,), jnp.float32),
            pltpu.VMEM((_NS, _KK // _NS), jnp.float32),
            pltpu.VMEM((_KK // _NS,), jnp.float32),
            pltpu.VMEM((_NS, (_K * _D) // _NS), jnp.float32),
            pltpu.VMEM((4, _D), jnp.float32),
            pltpu.VMEM((_KK,), jnp.float32),
            pltpu.VMEM((_NEI + 16,), jnp.int32),
            pltpu.VMEM((2, _NEI), jnp.int32),
            pltpu.VMEM((_NS, 3 * _K), jnp.float32),
            pltpu.VMEM((2 * _K,), jnp.float32),
            pltpu.VMEM_SHARED((_NS, _KK), jnp.float32),
            pltpu.VMEM_SHARED((_KK,), jnp.float32),
            pltpu.VMEM_SHARED((_NS, _K * _D), jnp.float32),
            pltpu.VMEM_SHARED((_NS, 3 * _K), jnp.float32),
            pltpu.SemaphoreType.DMA,
        ],
    )
    return f(xc, px, py, ei)


@jax.jit
def kernel(x, edge_index, pos, batch, W, b):
    ei = edge_index.astype(jnp.int32)
    px = pos[:, 0] + 0.0
    py = pos[:, 1] + 0.0
    soutm, souts, sout1 = _edge_stage(x, ei, px, py)
    wt = W.T.reshape(4 * _D, _D)
    b2 = b.reshape(1, _D)
    xc = _mlp_stage(soutm, souts, sout1, x, wt, b2)
    pooled_x, pposf, pooled_ei = _pool_stage(xc, px, py, ei)
    pooled_pos = pposf.reshape(_K, 2)
    pooled_batch = jnp.zeros((_K,), batch.dtype)
    return (pooled_x, pooled_pos, pooled_batch, pooled_ei, pos, batch)


# speedup vs baseline: 1.0131x; 1.0131x over previous
import functools
import jax
import jax.numpy as jnp
from jax import lax
from jax.experimental import pallas as pl
from jax.experimental.pallas import tpu as pltpu
from jax.experimental.pallas import tpu_sc as plsc

_N = 10000
_E = 320000
_D = 128
_NC = 2
_NS = 16
_CH = 80
_EPT = _E // _NS
_VOX = 8
_K = _VOX * _VOX
_KK = _K * _K
_NEI = _KK - _K


def _iota16():
    return lax.iota(jnp.int32, 16)


_ZR = 624
_BCH = 50
_BE = _BCH * _CH
_NBLK = _EPT // _BE


def _edge_body(x_hbm, ei_hbm, px_hbm, py_hbm, zm_hbm, zs_hbm,
               outm_hbm, outs_hbm, out1_hbm,
               srcb, dstb, wv,
               dstl0, dstl1, pjx0, pjy0, pix0, piy0, pjx1, pjy1, pix1, piy1,
               rows0, rows1, outs0, outs1,
               acc_sh, accs_sh, sg0, sg1, ss0, ss1):
    c = lax.axis_index("c")
    s = lax.axis_index("s")
    a0 = (1 - c).astype(jnp.float32)
    a1 = c.astype(jnp.float32)
    iot = _iota16()
    m_is0 = iot == 0
    sv1 = jnp.where(iot == 1, 1.0, 0.0).astype(jnp.float32)

    slots = (
        dict(dstl=dstl0, pjx=pjx0, pjy=pjy0, pix=pix0, piy=piy0,
             rows=rows0, outs=outs0, sg=sg0, ss=ss0),
        dict(dstl=dstl1, pjx=pjx1, pjy=pjy1, pix=pix1, piy=piy1,
             rows=rows1, outs=outs1, sg=sg1, ss=ss1),
    )

    def g_descs(kb, sl, pass1):
        srcs = srcb.at[pl.ds(kb * _CH, _CH)]
        dsts = dstb.at[pl.ds(kb * _CH, _CH)]
        ds_ = [(x_hbm.at[srcs], sl["rows"])]
        if pass1:
            ds_ += [(px_hbm.at[srcs], sl["pjx"]),
                    (py_hbm.at[srcs], sl["pjy"]),
                    (px_hbm.at[dsts], sl["pix"]),
                    (py_hbm.at[dsts], sl["piy"])]
        return ds_

    def fire_gathers(kb, sl, pass1):
        for src, dst in g_descs(kb, sl, pass1):
            pltpu.async_copy(src, dst, sl["sg"])

    def wait_gathers(kb, sl, pass1):
        for src, dst in g_descs(kb, sl, pass1):
            pltpu.make_async_copy(src, dst, sl["sg"]).wait()

    def fire_scatter(sl, pass1):
        pltpu.async_copy(sl["rows"], acc_sh.at[sl["dstl"]], sl["ss"],
                         add=True)
        if pass1:
            pltpu.async_copy(sl["outs"], accs_sh.at[sl["dstl"]], sl["ss"],
                             add=True)

    def wait_scatter(sl, pass1):
        pltpu.make_async_copy(sl["rows"], acc_sh.at[sl["dstl"]],
                              sl["ss"]).wait()
        if pass1:
            pltpu.make_async_copy(sl["outs"], accs_sh.at[sl["dstl"]],
                                  sl["ss"]).wait()

    def compute_chunk(kb, sl):
        for g in range(_CH // 16):
            vsl = pl.ds(g * 16, 16)
            dx = sl["pix"][vsl] - sl["pjx"][vsl]
            dy = sl["piy"][vsl] - sl["pjy"][vsl]
            sc = 1.0 / (dx * dx + dy * dy + 0.01)
            wv[vsl] = (dx * a0 + dy * a1) * sc
            sl["dstl"][vsl] = dstb[pl.ds(kb * _CH + g * 16, 16)]

        rows, outs = sl["rows"], sl["outs"]

        def edge(e, _):
            wf = plsc.load_gather(wv, [jnp.broadcast_to(e, (16,))])
            for j in range(_D // 16):
                rows[e, pl.ds(j * 16, 16)] = wf * rows[e, pl.ds(j * 16, 16)]
            outs[e, pl.ds(0, 16)] = jnp.where(m_is0, wf, sv1)
            return 0

        lax.fori_loop(0, _CH, edge, 0)

    def copy_dstl(kb, sl):
        for g in range(_CH // 16):
            vsl = pl.ds(g * 16, 16)
            sl["dstl"][vsl] = dstb[pl.ds(kb * _CH + g * 16, 16)]

    def run_pass(pass1):
        pltpu.sync_copy(zm_hbm, acc_sh.at[pl.ds(s * _ZR, _ZR)])
        if pass1:
            pltpu.sync_copy(zs_hbm, accs_sh.at[pl.ds(s * _ZR, _ZR)])

        @pl.when(s == _NS - 1)
        def ztail():
            pltpu.sync_copy(zm_hbm.at[pl.ds(0, 16)],
                            acc_sh.at[pl.ds(_NS * _ZR, 16)])
            if pass1:
                pltpu.sync_copy(zs_hbm.at[pl.ds(0, 16)],
                                accs_sh.at[pl.ds(_NS * _ZR, 16)])
        plsc.subcore_barrier()

        for b in range(_NBLK):
            base = s * _EPT + b * _BE
            pltpu.sync_copy(ei_hbm.at[0, pl.ds(base, _BE)], srcb)
            pltpu.sync_copy(ei_hbm.at[1, pl.ds(base, _BE)], dstb)
            fire_gathers(0, slots[0], pass1)

            def pair(i2, _):
                for par in (0, 1):
                    kb = i2 * 2 + par
                    sl = slots[par]
                    nxt = slots[1 - par]
                    if par == 0:
                        @pl.when(i2 > 0)
                        def wn():
                            wait_scatter(nxt, pass1)
                        fire_gathers(kb + 1, nxt, pass1)
                    else:
                        wait_scatter(nxt, pass1)

                        @pl.when(i2 < _BCH // 2 - 1)
                        def pf():
                            fire_gathers(kb + 1, nxt, pass1)
                    wait_gathers(kb, sl, pass1)
                    if pass1:
                        compute_chunk(kb, sl)
                    else:
                        copy_dstl(kb, sl)
                    fire_scatter(sl, pass1)
                return 0

            lax.fori_loop(0, _BCH // 2, pair, 0)
            wait_scatter(slots[1], pass1)
        plsc.subcore_barrier()

    run_pass(True)
    obase = c * _N
    pltpu.sync_copy(acc_sh.at[pl.ds(s * _ZR, _ZR)],
                    outm_hbm.at[pl.ds(obase + s * _ZR, _ZR)])
    pltpu.sync_copy(accs_sh.at[pl.ds(s * _ZR, _ZR)],
                    outs_hbm.at[pl.ds(obase + s * _ZR, _ZR)])

    @pl.when(s == _NS - 1)
    def otail1():
        pltpu.sync_copy(acc_sh.at[pl.ds(_NS * _ZR, 16)],
                        outm_hbm.at[pl.ds(obase + _NS * _ZR, 16)])
        pltpu.sync_copy(accs_sh.at[pl.ds(_NS * _ZR, 16)],
                        outs_hbm.at[pl.ds(obase + _NS * _ZR, 16)])
    plsc.subcore_barrier()

    run_pass(False)

    @pl.when(c == 0)
    def otail2():
        pltpu.sync_copy(acc_sh.at[pl.ds(s * _ZR, _ZR)],
                        out1_hbm.at[pl.ds(s * _ZR, _ZR)])

        @pl.when(s == _NS - 1)
        def otail2t():
            pltpu.sync_copy(acc_sh.at[pl.ds(_NS * _ZR, 16)],
                            out1_hbm.at[pl.ds(_NS * _ZR, 16)])


def _edge_stage(x, ei, px, py):
    mesh = plsc.VectorSubcoreMesh(core_axis_name="c", subcore_axis_name="s",
                                  num_cores=_NC, num_subcores=_NS)
    zm = jnp.zeros((_ZR, _D), jnp.float32)
    zs = jnp.zeros((_ZR, 16), jnp.float32)
    f = pl.kernel(
        _edge_body,
        out_type=(
            jax.ShapeDtypeStruct((_NC * _N, _D), jnp.float32),
            jax.ShapeDtypeStruct((_NC * _N, 16), jnp.float32),
            jax.ShapeDtypeStruct((_N, _D), jnp.float32),
        ),
        mesh=mesh,
        compiler_params=pltpu.CompilerParams(
            use_tc_tiling_on_sc=False, needs_layout_passes=False),
        scratch_types=[
            pltpu.VMEM((_BE,), jnp.int32),
            pltpu.VMEM((_BE,), jnp.int32),
            pltpu.VMEM((_CH,), jnp.float32),
            pltpu.VMEM((_CH,), jnp.int32),
            pltpu.VMEM((_CH,), jnp.int32),
            pltpu.VMEM((_CH,), jnp.float32),
            pltpu.VMEM((_CH,), jnp.float32),
            pltpu.VMEM((_CH,), jnp.float32),
            pltpu.VMEM((_CH,), jnp.float32),
            pltpu.VMEM((_CH,), jnp.float32),
            pltpu.VMEM((_CH,), jnp.float32),
            pltpu.VMEM((_CH,), jnp.float32),
            pltpu.VMEM((_CH,), jnp.float32),
            pltpu.VMEM((_CH, _D), jnp.float32),
            pltpu.VMEM((_CH, _D), jnp.float32),
            pltpu.VMEM((_CH, 16), jnp.float32),
            pltpu.VMEM((_CH, 16), jnp.float32),
            pltpu.VMEM_SHARED((_N, _D), jnp.float32),
            pltpu.VMEM_SHARED((_N, 16), jnp.float32),
            pltpu.SemaphoreType.DMA,
            pltpu.SemaphoreType.DMA,
            pltpu.SemaphoreType.DMA,
            pltpu.SemaphoreType.DMA,
        ],
    )
    return f(x, ei, px, py, zm, zs)


def _mlp_body(sa_ref, sb_ref, c0_ref, c1_ref, s1_ref, x_ref, wt_ref, b_ref,
              out_ref):
    sa = sa_ref[...]
    sb = sb_ref[...]
    s1c = s1_ref[...]
    xb = x_ref[...]
    suma = c0_ref[:, 0:1]
    sumb = c1_ref[:, 0:1]
    cnt = jnp.maximum(c0_ref[:, 1:2], 1.0)
    prop = jnp.concatenate(
        [(xb * suma - sa) / cnt, (xb * sumb - sb) / cnt, s1c / cnt, xb], axis=1)
    acc = jnp.dot(prop, wt_ref[...], preferred_element_type=jnp.float32)
    out_ref[...] = jnp.maximum(acc + b_ref[...], 0.0)


def _mlp_stage(soutm, souts, sout1, x, wt, b2):
    blk = 1000
    grid = _N // blk
    return pl.pallas_call(
        _mlp_body,
        grid=(grid,),
        in_specs=[
            pl.BlockSpec((blk, _D), lambda i: (i, 0)),
            pl.BlockSpec((blk, _D), lambda i: (grid + i, 0)),
            pl.BlockSpec((blk, 16), lambda i: (i, 0)),
            pl.BlockSpec((blk, 16), lambda i: (grid + i, 0)),
            pl.BlockSpec((blk, _D), lambda i: (i, 0)),
            pl.BlockSpec((blk, _D), lambda i: (i, 0)),
            pl.BlockSpec((4 * _D, _D), lambda i: (0, 0)),
            pl.BlockSpec((1, _D), lambda i: (0, 0)),
        ],
        out_specs=pl.BlockSpec((blk, _D), lambda i: (i, 0)),
        out_shape=jax.ShapeDtypeStruct((_N, _D), jnp.float32),
    )(soutm, soutm, souts, souts, sout1, x, wt, b2)


def _pool_body(xc_hbm, px_hbm, py_hbm, ei_hbm, outx_hbm, outpos_hbm,
               outei_hbm, posx, posy, xcb, posxc, posyc, srcv, dstv, occ,
               maxf, pkl, stripv, redv, stripx, redx, occv, codesb, eib,
               pkv, pposb, occ_sh, occr_sh, max_sh, pk_sh, sem):
    c = lax.axis_index("c")
    s = lax.axis_index("s")
    iot = _iota16()
    onesf = jnp.full((16,), 1.0, jnp.float32)
    m0 = iot == 0

    def vox(p):
        return jnp.clip((p * 8.0).astype(jnp.int32), 0, _VOX - 1)

    @pl.when(c == 0)
    def occupancy():
        pltpu.sync_copy(px_hbm, posx)
        pltpu.sync_copy(py_hbm, posy)

        def z(g, _):
            occ[pl.ds(g * 16, 16)] = jnp.zeros((16,), jnp.float32)
            return 0
        lax.fori_loop(0, _KK // 16, z, 0)

        for b in range(_EPT // _BE):
            base = s * _EPT + b * _BE
            pltpu.sync_copy(ei_hbm.at[0, pl.ds(base, _BE)], srcv)
            pltpu.sync_copy(ei_hbm.at[1, pl.ds(base, _BE)], dstv)

            def grp(g, _):
                vsl = pl.ds(g * 16, 16)
                s16 = srcv[vsl]
                d16 = dstv[vsl]
                cs = vox(plsc.load_gather(posx, [s16])) * _VOX + \
                    vox(plsc.load_gather(posy, [s16]))
                cd = vox(plsc.load_gather(posx, [d16])) * _VOX + \
                    vox(plsc.load_gather(posy, [d16]))
                code = cs * _K + cd
                plsc.store_scatter(occ, [code], onesf, mask=cs != cd)
                return 0
            lax.fori_loop(0, _BE // 16, grp, 0)

        pltpu.sync_copy(occ, occ_sh.at[s])
        plsc.subcore_barrier()
        strip = _KK // _NS
        pltpu.sync_copy(occ_sh.at[:, pl.ds(s * strip, strip)], stripv)
        for k in range(strip // 16):
            accv = stripv[0, pl.ds(k * 16, 16)]
            for r in range(1, _NS):
                accv = jnp.maximum(accv, stripv[r, pl.ds(k * 16, 16)])
            redv[pl.ds(k * 16, 16)] = accv
        pltpu.sync_copy(redv, occr_sh.at[pl.ds(s * strip, strip)])
        plsc.subcore_barrier()

        @pl.when(s == 0)
        def compact():
            pltpu.sync_copy(occr_sh, occv)

            def fill(g, _):
                codesb[pl.ds(g * 16, 16)] = jnp.full((16,), _KK, jnp.int32)
                return 0
            lax.fori_loop(0, (_NEI + 16) // 16, fill, 0)

            def step(g, off):
                ov = occv[pl.ds(g * 16, 16)]
                m = ov > 0.0
                codes = jnp.broadcast_to(g * 16, (16,)) + iot
                plsc.store_compressed(codesb.at[pl.ds(off, 16)], codes, mask=m)
                return off + jnp.max(plsc.all_reduce_population_count(m))
            lax.fori_loop(0, _KK // 16, step, 0)

            def emit(g, _):
                cv = codesb[pl.ds(g * 16, 16)]
                eib[0, pl.ds(g * 16, 16)] = lax.shift_right_arithmetic(cv, 6)
                eib[1, pl.ds(g * 16, 16)] = lax.bitwise_and(cv, 63)
                return 0
            lax.fori_loop(0, _NEI // 16, emit, 0)
            pltpu.sync_copy(eib, outei_hbm)

    @pl.when(c == 1)
    def pooling():
        def ini(g, _):
            maxf[pl.ds(g * 16, 16)] = jnp.full((16,), -jnp.inf, jnp.float32)
            return 0
        lax.fori_loop(0, (_K * _D) // 16, ini, 0)
        for k in range(192 // 16):
            pkl[pl.ds(k * 16, 16)] = jnp.zeros((16,), jnp.float32)

        def do_chunk(base, nrows):
            pltpu.sync_copy(xc_hbm.at[pl.ds(base, nrows)],
                            xcb.at[pl.ds(0, nrows)])
            pltpu.sync_copy(px_hbm.at[pl.ds(base, nrows)],
                            posxc.at[pl.ds(0, nrows)])
            pltpu.sync_copy(py_hbm.at[pl.ds(base, nrows)],
                            posyc.at[pl.ds(0, nrows)])

            def row(r, _):
                rsp = jnp.broadcast_to(r, (16,))
                px = plsc.load_gather(posxc, [rsp])
                py = plsc.load_gather(posyc, [rsp])
                code = vox(px) * _VOX + vox(py)
                for j in range(_D // 16):
                    idx = code * _D + j * 16 + iot
                    cur = plsc.load_gather(maxf, [idx])
                    xv = xcb[r, pl.ds(j * 16, 16)]
                    plsc.store_scatter(maxf, [idx], jnp.maximum(cur, xv))
                plsc.addupdate_scatter(pkl, [code], px, mask=m0)
                plsc.addupdate_scatter(pkl, [code + _K], py, mask=m0)
                plsc.addupdate_scatter(pkl, [code + 2 * _K], onesf, mask=m0)
                return 0
            lax.fori_loop(0, nrows, row, 0)

        for j in range(6):
            do_chunk(s * 624 + j * 104, 104)

        @pl.when(s == 0)
        def rtail():
            do_chunk(_NS * 624, 16)

        pltpu.sync_copy(maxf, max_sh.at[s])
        pltpu.sync_copy(pkl, pk_sh.at[s])
        plsc.subcore_barrier()
        strip = (_K * _D) // _NS
        pltpu.sync_copy(max_sh.at[:, pl.ds(s * strip, strip)], stripx)
        for k in range(strip // 16):
            accv = stripx[0, pl.ds(k * 16, 16)]
            for r in range(1, _NS):
                accv = jnp.maximum(accv, stripx[r, pl.ds(k * 16, 16)])
            redx[k // 8, pl.ds((k % 8) * 16, 16)] = accv
        pltpu.sync_copy(redx, outx_hbm.at[pl.ds(s * 4, 4)])

        @pl.when(s == 15)
        def posfin():
            pltpu.sync_copy(pk_sh, pkv)
            for k in range(4):
                pxv = pkv[0, pl.ds(k * 16, 16)]
                pyv = pkv[0, pl.ds(_K + k * 16, 16)]
                cnv = pkv[0, pl.ds(2 * _K + k * 16, 16)]
                for r in range(1, _NS):
                    pxv = pxv + pkv[r, pl.ds(k * 16, 16)]
                    pyv = pyv + pkv[r, pl.ds(_K + k * 16, 16)]
                    cnv = cnv + pkv[r, pl.ds(2 * _K + k * 16, 16)]
                cnv = jnp.maximum(cnv, 1.0)
                ridx = (iot + k * 16) * 2
                plsc.store_scatter(pposb, [ridx], pxv / cnv)
                plsc.store_scatter(pposb, [ridx + 1], pyv / cnv)
            pltpu.sync_copy(pposb, outpos_hbm)


def _pool_stage(xc, px, py, ei):
    mesh = plsc.VectorSubcoreMesh(core_axis_name="c", subcore_axis_name="s",
                                  num_cores=_NC, num_subcores=_NS)
    f = pl.kernel(
        _pool_body,
        compiler_params=pltpu.CompilerParams(
            use_tc_tiling_on_sc=False, needs_layout_passes=False),
        out_type=(
            jax.ShapeDtypeStruct((_K, _D), jnp.float32),
            jax.ShapeDtypeStruct((2 * _K,), jnp.float32),
            jax.ShapeDtypeStruct((2, _NEI), jnp.int32),
        ),
        mesh=mesh,
        scratch_types=[
            pltpu.VMEM((_N,), jnp.float32),
            pltpu.VMEM((_N,), jnp.float32),
            pltpu.VMEM((104, _D), jnp.float32),
            pltpu.VMEM((104,), jnp.float32),
            pltpu.VMEM((104,), jnp.float32),
            pltpu.VMEM((_BE,), jnp.int32),
            pltpu.VMEM((_BE,), jnp.int32),
            pltpu.VMEM((_KK,), jnp.float32),
            pltpu.VMEM((_K * _D,), jnp.float32),
            pltpu.VMEM((3 * _K,), jnp.float32),
            pltpu.VMEM((_NS, _KK // _NS), jnp.float32),
            pltpu.VMEM((_KK // _NS,), jnp.float32),
            pltpu.VMEM((_NS, (_K * _D) // _NS), jnp.float32),
            pltpu.VMEM((4, _D), jnp.float32),
            pltpu.VMEM((_KK,), jnp.float32),
            pltpu.VMEM((_NEI + 16,), jnp.int32),
            pltpu.VMEM((2, _NEI), jnp.int32),
            pltpu.VMEM((_NS, 3 * _K), jnp.float32),
            pltpu.VMEM((2 * _K,), jnp.float32),
            pltpu.VMEM_SHARED((_NS, _KK), jnp.float32),
            pltpu.VMEM_SHARED((_KK,), jnp.float32),
            pltpu.VMEM_SHARED((_NS, _K * _D), jnp.float32),
            pltpu.VMEM_SHARED((_NS, 3 * _K), jnp.float32),
            pltpu.SemaphoreType.DMA,
        ],
    )
    return f(xc, px, py, ei)


@jax.jit
def kernel(x, edge_index, pos, batch, W, b):
    ei = edge_index.astype(jnp.int32)
    px = pos[:, 0] + 0.0
    py = pos[:, 1] + 0.0
    soutm, souts, sout1 = _edge_stage(x, ei, px, py)
    wt = W.T.reshape(4 * _D, _D)
    b2 = b.reshape(1, _D)
    xc = _mlp_stage(soutm, souts, sout1, x, wt, b2)
    pooled_x, pposf, pooled_ei = _pool_stage(xc, px, py, ei)
    pooled_pos = pposf.reshape(_K, 2)
    pooled_batch = jnp.zeros((_K,), batch.dtype)
    return (pooled_x, pooled_pos, pooled_batch, pooled_ei, pos, batch)
